# baseline (device time: 25568 ns/iter reference)
import jax
import jax.numpy as jnp
from jax import lax
from jax.experimental import pallas as pl
from jax.experimental.pallas import tpu as pltpu

N_DEV = 4
B = 2
SQ = 256
SKV = 256
HQ_PER = 4
DH = 64
DM = 512


def kernel(x, Wq, K_ext, V_ext, Wo):

    def body(x_hbm, wq_hbm, k_hbm, v_hbm, wo_hbm, out_ref,
             xv, wqv, kv, vv, wov,
             send_a, recv_a, send_b, recv_b,
             load_sems, send_sems_a, recv_sems_a, send_sems_b, recv_sems_b):
        my_i = lax.axis_index("i")
        left = (my_i - 1) % N_DEV
        right = (my_i + 1) % N_DEV
        partner_a = my_i ^ 1
        partner_b = 3 - my_i
        h0 = my_i * HQ_PER

        loads = [
            pltpu.make_async_copy(x_hbm, xv, load_sems.at[0]),
            pltpu.make_async_copy(wq_hbm, wqv, load_sems.at[1]),
            pltpu.make_async_copy(
                k_hbm.at[:, :, pl.ds(h0, HQ_PER), :], kv, load_sems.at[2]),
            pltpu.make_async_copy(
                v_hbm.at[:, :, pl.ds(h0, HQ_PER), :], vv, load_sems.at[3]),
            pltpu.make_async_copy(wo_hbm, wov, load_sems.at[4]),
        ]
        for c in loads:
            c.start()

        barrier = pltpu.get_barrier_semaphore()
        for nbr in (left, right):
            pl.semaphore_signal(
                barrier, inc=1,
                device_id=(nbr,), device_id_type=pl.DeviceIdType.MESH,
            )
        pl.semaphore_wait(barrier, 2)

        for c in loads:
            c.wait()

        wq = (wqv[...] * 0.125).astype(jnp.bfloat16)
        wo = wov[...].astype(jnp.bfloat16)

        def softmax_ctx(q, k, v):
            s = lax.dot_general(
                q, k, (((1,), (1,)), ((), ())),
                preferred_element_type=jnp.float32,
            )
            w = jnp.exp(s)
            r = 1.0 / jnp.sum(w, axis=-1, keepdims=True)
            return jnp.dot((w * r).astype(jnp.bfloat16), v,
                           preferred_element_type=jnp.float32)

        def compute_batch(b):
            xb = xv[b].astype(jnp.bfloat16)
            qf = jnp.dot(xb, wq, preferred_element_type=jnp.float32)
            ctx_blocks = []
            for h in range(HQ_PER):
                qh = qf[:, h * DH:(h + 1) * DH].astype(jnp.bfloat16)
                kh = kv[b, :, h, :].astype(jnp.bfloat16)
                vh = vv[b, :, h, :].astype(jnp.bfloat16)
                ctx_a = softmax_ctx(qh[64:192], kh[0:192], vh[0:192])
                qg = jnp.concatenate([qh[0:64], qh[192:256]], axis=0)
                kg = jnp.concatenate([kh[0:64], kh[192:256]], axis=0)
                vg = jnp.concatenate([vh[0:64], vh[192:256]], axis=0)
                ctx_b = softmax_ctx(qg, kg, vg)
                ctx_blocks.append(jnp.concatenate(
                    [ctx_b[0:64], ctx_a, ctx_b[64:128]], axis=0,
                ).astype(jnp.bfloat16))
            ctx_full = jnp.concatenate(ctx_blocks, axis=1)
            return jnp.dot(ctx_full, wo, preferred_element_type=jnp.float32)

        def exchange(phase_send, phase_recv, ssems, rsems, partner, b):
            return pltpu.make_async_remote_copy(
                src_ref=phase_send.at[b],
                dst_ref=phase_recv.at[b],
                send_sem=ssems.at[b],
                recv_sem=rsems.at[b],
                device_id=(partner,),
                device_id_type=pl.DeviceIdType.MESH,
            )

        rdma_a = [None, None]
        rdma_b = [None, None]
        for b in range(B):
            acc = compute_batch(b)
            out_ref[b] = acc
            send_a[b] = acc.astype(jnp.bfloat16)
            rdma_a[b] = exchange(send_a, recv_a, send_sems_a, recv_sems_a,
                                 partner_a, b)
            rdma_a[b].start()

        for b in range(B):
            rdma_a[b].wait()
            pair_sum = out_ref[b] + recv_a[b].astype(jnp.float32)
            out_ref[b] = pair_sum
            send_b[b] = pair_sum.astype(jnp.bfloat16)
            rdma_b[b] = exchange(send_b, recv_b, send_sems_b, recv_sems_b,
                                 partner_b, b)
            rdma_b[b].start()

        for b in range(B):
            rdma_b[b].wait()
            out_ref[b] = out_ref[b] + recv_b[b].astype(jnp.float32)

    comm = pltpu.VMEM((B, SQ, DM), jnp.bfloat16)
    return pl.pallas_call(
        body,
        out_shape=jax.ShapeDtypeStruct((B, SQ, DM), jnp.float32),
        in_specs=[pl.BlockSpec(memory_space=pl.ANY)] * 5,
        out_specs=pl.BlockSpec(memory_space=pltpu.VMEM),
        scratch_shapes=[
            pltpu.VMEM((B, SQ, DM), jnp.float32),
            pltpu.VMEM((DM, HQ_PER * DH), jnp.float32),
            pltpu.VMEM((B, SKV, HQ_PER, DH), jnp.float32),
            pltpu.VMEM((B, SKV, HQ_PER, DH), jnp.float32),
            pltpu.VMEM((HQ_PER * DH, DM), jnp.float32),
            comm, comm, comm, comm,
            pltpu.SemaphoreType.DMA((5,)),
            pltpu.SemaphoreType.DMA((B,)),
            pltpu.SemaphoreType.DMA((B,)),
            pltpu.SemaphoreType.DMA((B,)),
            pltpu.SemaphoreType.DMA((B,)),
        ],
        compiler_params=pltpu.CompilerParams(collective_id=0),
    )(x, Wq, K_ext, V_ext, Wo)


# device time: 20035 ns/iter; 1.2762x vs baseline; 1.2762x over previous
import jax
import jax.numpy as jnp
from jax import lax
from jax.experimental import pallas as pl
from jax.experimental.pallas import tpu as pltpu

N_DEV = 4
B = 2
SQ = 256
SKV = 256
HQ_PER = 4
DH = 64
DM = 512


def kernel(x, Wq, K_ext, V_ext, Wo):
    my = lax.axis_index("i")
    k_loc = lax.dynamic_slice_in_dim(
        K_ext, my * HQ_PER, HQ_PER, axis=2).reshape(B, SKV, HQ_PER * DH)
    v_loc = lax.dynamic_slice_in_dim(
        V_ext, my * HQ_PER, HQ_PER, axis=2).reshape(B, SKV, HQ_PER * DH)

    def body(x_hbm, wq_hbm, k_hbm, v_hbm, wo_hbm, out_ref,
             xv, wqv, kv, vv, wov,
             send_a, recv_a, send_b, recv_b,
             load_sems, send_sems_a, recv_sems_a, send_sems_b, recv_sems_b):
        my_i = lax.axis_index("i")
        left = (my_i - 1) % N_DEV
        right = (my_i + 1) % N_DEV
        partner_a = my_i ^ 1
        partner_b = 3 - my_i

        loads = [
            pltpu.make_async_copy(x_hbm, xv, load_sems.at[0]),
            pltpu.make_async_copy(wq_hbm, wqv, load_sems.at[1]),
            pltpu.make_async_copy(k_hbm, kv, load_sems.at[2]),
            pltpu.make_async_copy(v_hbm, vv, load_sems.at[3]),
            pltpu.make_async_copy(wo_hbm, wov, load_sems.at[4]),
        ]
        for c in loads:
            c.start()

        barrier = pltpu.get_barrier_semaphore()
        for nbr in (left, right):
            pl.semaphore_signal(
                barrier, inc=1,
                device_id=(nbr,), device_id_type=pl.DeviceIdType.MESH,
            )
        pl.semaphore_wait(barrier, 2)

        for c in loads:
            c.wait()

        wq = (wqv[...] * 0.125).astype(jnp.bfloat16)
        wo = wov[...].astype(jnp.bfloat16)

        def softmax_ctx(q, k, v):
            s = lax.dot_general(
                q, k, (((1,), (1,)), ((), ())),
                preferred_element_type=jnp.float32,
            )
            w = jnp.exp(s)
            r = 1.0 / jnp.sum(w, axis=-1, keepdims=True)
            return jnp.dot((w * r).astype(jnp.bfloat16), v,
                           preferred_element_type=jnp.float32)

        def compute_batch(b):
            xb = xv[b].astype(jnp.bfloat16)
            qf = jnp.dot(xb, wq, preferred_element_type=jnp.float32)
            ctx_blocks = []
            for h in range(HQ_PER):
                qh = qf[:, h * DH:(h + 1) * DH].astype(jnp.bfloat16)
                kh = kv[b][:, h * DH:(h + 1) * DH].astype(jnp.bfloat16)
                vh = vv[b][:, h * DH:(h + 1) * DH].astype(jnp.bfloat16)
                ctx_a = softmax_ctx(qh[64:192], kh[0:192], vh[0:192])
                qg = jnp.concatenate([qh[0:64], qh[192:256]], axis=0)
                kg = jnp.concatenate([kh[0:64], kh[192:256]], axis=0)
                vg = jnp.concatenate([vh[0:64], vh[192:256]], axis=0)
                ctx_b = softmax_ctx(qg, kg, vg)
                ctx_blocks.append(jnp.concatenate(
                    [ctx_b[0:64], ctx_a, ctx_b[64:128]], axis=0,
                ).astype(jnp.bfloat16))
            ctx_full = jnp.concatenate(ctx_blocks, axis=1)
            return jnp.dot(ctx_full, wo, preferred_element_type=jnp.float32)

        def exchange(phase_send, phase_recv, ssems, rsems, partner, b):
            return pltpu.make_async_remote_copy(
                src_ref=phase_send.at[b],
                dst_ref=phase_recv.at[b],
                send_sem=ssems.at[b],
                recv_sem=rsems.at[b],
                device_id=(partner,),
                device_id_type=pl.DeviceIdType.MESH,
            )

        rdma_a = [None, None]
        rdma_b = [None, None]
        for b in range(B):
            acc = compute_batch(b)
            out_ref[b] = acc
            send_a[b] = acc.astype(jnp.bfloat16)
            rdma_a[b] = exchange(send_a, recv_a, send_sems_a, recv_sems_a,
                                 partner_a, b)
            rdma_a[b].start()

        for b in range(B):
            rdma_a[b].wait()
            pair_sum = out_ref[b] + recv_a[b].astype(jnp.float32)
            out_ref[b] = pair_sum
            send_b[b] = pair_sum.astype(jnp.bfloat16)
            rdma_b[b] = exchange(send_b, recv_b, send_sems_b, recv_sems_b,
                                 partner_b, b)
            rdma_b[b].start()

        for b in range(B):
            rdma_b[b].wait()
            out_ref[b] = out_ref[b] + recv_b[b].astype(jnp.float32)

    comm = pltpu.VMEM((B, SQ, DM), jnp.bfloat16)
    return pl.pallas_call(
        body,
        out_shape=jax.ShapeDtypeStruct((B, SQ, DM), jnp.float32),
        in_specs=[pl.BlockSpec(memory_space=pl.ANY)] * 5,
        out_specs=pl.BlockSpec(memory_space=pltpu.VMEM),
        scratch_shapes=[
            pltpu.VMEM((B, SQ, DM), jnp.float32),
            pltpu.VMEM((DM, HQ_PER * DH), jnp.float32),
            pltpu.VMEM((B, SKV, HQ_PER * DH), jnp.float32),
            pltpu.VMEM((B, SKV, HQ_PER * DH), jnp.float32),
            pltpu.VMEM((HQ_PER * DH, DM), jnp.float32),
            comm, comm, comm, comm,
            pltpu.SemaphoreType.DMA((5,)),
            pltpu.SemaphoreType.DMA((B,)),
            pltpu.SemaphoreType.DMA((B,)),
            pltpu.SemaphoreType.DMA((B,)),
            pltpu.SemaphoreType.DMA((B,)),
        ],
        compiler_params=pltpu.CompilerParams(collective_id=0),
    )(x, Wq, k_loc, v_loc, Wo)


# device time: 19110 ns/iter; 1.3379x vs baseline; 1.0484x over previous
import jax
import jax.numpy as jnp
from jax import lax
from jax.experimental import pallas as pl
from jax.experimental.pallas import tpu as pltpu

N_DEV = 4
B = 2
SQ = 256
SKV = 256
HQ_PER = 4
DH = 64
DM = 512


def kernel(x, Wq, K_ext, V_ext, Wo):
    my = lax.axis_index("i")
    k_loc = lax.dynamic_slice_in_dim(
        K_ext, my * HQ_PER, HQ_PER, axis=2).reshape(B, SKV, HQ_PER * DH)
    v_loc = lax.dynamic_slice_in_dim(
        V_ext, my * HQ_PER, HQ_PER, axis=2).reshape(B, SKV, HQ_PER * DH)

    def body(x_hbm, wq_hbm, k_hbm, v_hbm, wo_hbm, out_ref,
             xv, wqv, kv, vv, wov,
             send_a, recv_a, send_b, recv_b,
             load_sems, send_sems_a, recv_sems_a, send_sems_b, recv_sems_b):
        my_i = lax.axis_index("i")
        left = (my_i - 1) % N_DEV
        right = (my_i + 1) % N_DEV
        partner_a = my_i ^ 1
        partner_b = 3 - my_i

        loads = [
            pltpu.make_async_copy(x_hbm, xv, load_sems.at[0]),
            pltpu.make_async_copy(wq_hbm, wqv, load_sems.at[1]),
            pltpu.make_async_copy(k_hbm, kv, load_sems.at[2]),
            pltpu.make_async_copy(v_hbm, vv, load_sems.at[3]),
            pltpu.make_async_copy(wo_hbm, wov, load_sems.at[4]),
        ]
        for c in loads:
            c.start()

        barrier = pltpu.get_barrier_semaphore()
        for nbr in (left, right):
            pl.semaphore_signal(
                barrier, inc=1,
                device_id=(nbr,), device_id_type=pl.DeviceIdType.MESH,
            )
        pl.semaphore_wait(barrier, 2)

        for c in loads:
            c.wait()

        wq = (wqv[...] * 0.125).astype(jnp.bfloat16)
        wo = wov[...].astype(jnp.bfloat16)

        def softmax_ctx(q, k, v):
            s = lax.dot_general(
                q, k, (((1,), (1,)), ((), ())),
                preferred_element_type=jnp.float32,
            )
            w = jnp.exp(s)
            r = 1.0 / jnp.sum(w, axis=-1, keepdims=True)
            return jnp.dot((w * r).astype(jnp.bfloat16), v,
                           preferred_element_type=jnp.float32)

        def compute_batch(b):
            xb = xv[b].astype(jnp.bfloat16)
            qf = jnp.dot(xb, wq, preferred_element_type=jnp.float32)
            ctx_blocks = []
            for h in range(HQ_PER):
                qh = qf[:, h * DH:(h + 1) * DH].astype(jnp.bfloat16)
                kh = kv[b][:, h * DH:(h + 1) * DH].astype(jnp.bfloat16)
                vh = vv[b][:, h * DH:(h + 1) * DH].astype(jnp.bfloat16)
                ctx_a = softmax_ctx(qh[64:192], kh[0:192], vh[0:192])
                qg = jnp.concatenate([qh[0:64], qh[192:256]], axis=0)
                kg = jnp.concatenate([kh[0:64], kh[192:256]], axis=0)
                vg = jnp.concatenate([vh[0:64], vh[192:256]], axis=0)
                ctx_b = softmax_ctx(qg, kg, vg)
                ctx_blocks.append(jnp.concatenate(
                    [ctx_b[0:64], ctx_a, ctx_b[64:128]], axis=0,
                ).astype(jnp.bfloat16))
            ctx_full = jnp.concatenate(ctx_blocks, axis=1)
            return jnp.dot(ctx_full, wo, preferred_element_type=jnp.float32)

        def exchange(phase_send, phase_recv, ssems, rsems, partner, b):
            return pltpu.make_async_remote_copy(
                src_ref=phase_send.at[b],
                dst_ref=phase_recv.at[b],
                send_sem=ssems.at[b],
                recv_sem=rsems.at[b],
                device_id=(partner,),
                device_id_type=pl.DeviceIdType.MESH,
            )

        phase1_partner = [partner_a, partner_b]
        phase2_partner = [partner_b, partner_a]
        rdma_1 = [None, None]
        rdma_2 = [None, None]
        for b in range(B):
            acc = compute_batch(b)
            out_ref[b] = acc
            send_a[b] = acc.astype(jnp.bfloat16)
            rdma_1[b] = exchange(send_a, recv_a, send_sems_a, recv_sems_a,
                                 phase1_partner[b], b)
            rdma_1[b].start()

        for b in range(B):
            rdma_1[b].wait()
            pair_sum = out_ref[b] + recv_a[b].astype(jnp.float32)
            out_ref[b] = pair_sum
            send_b[b] = pair_sum.astype(jnp.bfloat16)
            rdma_2[b] = exchange(send_b, recv_b, send_sems_b, recv_sems_b,
                                 phase2_partner[b], b)
            rdma_2[b].start()

        for b in range(B):
            rdma_2[b].wait()
            out_ref[b] = out_ref[b] + recv_b[b].astype(jnp.float32)

    comm = pltpu.VMEM((B, SQ, DM), jnp.bfloat16)
    return pl.pallas_call(
        body,
        out_shape=jax.ShapeDtypeStruct((B, SQ, DM), jnp.float32),
        in_specs=[pl.BlockSpec(memory_space=pl.ANY)] * 5,
        out_specs=pl.BlockSpec(memory_space=pltpu.VMEM),
        scratch_shapes=[
            pltpu.VMEM((B, SQ, DM), jnp.float32),
            pltpu.VMEM((DM, HQ_PER * DH), jnp.float32),
            pltpu.VMEM((B, SKV, HQ_PER * DH), jnp.float32),
            pltpu.VMEM((B, SKV, HQ_PER * DH), jnp.float32),
            pltpu.VMEM((HQ_PER * DH, DM), jnp.float32),
            comm, comm, comm, comm,
            pltpu.SemaphoreType.DMA((5,)),
            pltpu.SemaphoreType.DMA((B,)),
            pltpu.SemaphoreType.DMA((B,)),
            pltpu.SemaphoreType.DMA((B,)),
            pltpu.SemaphoreType.DMA((B,)),
        ],
        compiler_params=pltpu.CompilerParams(collective_id=0),
    )(x, Wq, k_loc, v_loc, Wo)


# device time: 16444 ns/iter; 1.5549x vs baseline; 1.1621x over previous
import jax
import jax.numpy as jnp
from jax import lax
from jax.experimental import pallas as pl
from jax.experimental.pallas import tpu as pltpu

N_DEV = 4
B = 2
SQ = 256
SKV = 256
HQ_PER = 4
DH = 64
DM = 512


def kernel(x, Wq, K_ext, V_ext, Wo):
    my = lax.axis_index("i")
    k_loc = lax.dynamic_slice_in_dim(
        K_ext, my * HQ_PER, HQ_PER, axis=2).reshape(B, SKV, HQ_PER * DH)
    v_loc = lax.dynamic_slice_in_dim(
        V_ext, my * HQ_PER, HQ_PER, axis=2).reshape(B, SKV, HQ_PER * DH)

    def body(x_hbm, wq_hbm, k_hbm, v_hbm, wo_hbm, out_ref,
             xv, wqv, kv, vv, wov,
             send_a, recv_a, send_b, recv_b,
             load_sems, send_sems_a, recv_sems_a, send_sems_b, recv_sems_b):
        my_i = lax.axis_index("i")
        left = (my_i - 1) % N_DEV
        right = (my_i + 1) % N_DEV
        partner_a = my_i ^ 1
        partner_b = 3 - my_i

        loads = [
            pltpu.make_async_copy(x_hbm, xv, load_sems.at[0]),
            pltpu.make_async_copy(wq_hbm, wqv, load_sems.at[1]),
            pltpu.make_async_copy(k_hbm, kv, load_sems.at[2]),
            pltpu.make_async_copy(v_hbm, vv, load_sems.at[3]),
            pltpu.make_async_copy(wo_hbm, wov, load_sems.at[4]),
        ]
        for c in loads:
            c.start()

        barrier = pltpu.get_barrier_semaphore()
        for nbr in (left, right):
            pl.semaphore_signal(
                barrier, inc=1,
                device_id=(nbr,), device_id_type=pl.DeviceIdType.MESH,
            )
        pl.semaphore_wait(barrier, 2)

        for c in loads:
            c.wait()

        wq = (wqv[...] * 0.125).astype(jnp.bfloat16)
        wo = wov[...].astype(jnp.bfloat16)

        def softmax_ctx(q, k, v):
            s = lax.dot_general(
                q, k, (((1,), (1,)), ((), ())),
                preferred_element_type=jnp.float32,
            )
            w = jnp.exp(s)
            r = 1.0 / jnp.sum(w, axis=-1, keepdims=True)
            return jnp.dot((w * r).astype(jnp.bfloat16), v,
                           preferred_element_type=jnp.float32)

        def compute_batch(b):
            xb = xv[b].astype(jnp.bfloat16)
            qf = jnp.dot(xb, wq, preferred_element_type=jnp.float32)
            ctx_blocks = []
            for h in range(HQ_PER):
                qh = qf[:, h * DH:(h + 1) * DH].astype(jnp.bfloat16)
                kh = kv[b][:, h * DH:(h + 1) * DH].astype(jnp.bfloat16)
                vh = vv[b][:, h * DH:(h + 1) * DH].astype(jnp.bfloat16)
                ctx_a = softmax_ctx(qh[64:192], kh[0:192], vh[0:192])
                qg = jnp.concatenate([qh[0:64], qh[192:256]], axis=0)
                kg = jnp.concatenate([kh[0:64], kh[192:256]], axis=0)
                vg = jnp.concatenate([vh[0:64], vh[192:256]], axis=0)
                ctx_b = softmax_ctx(qg, kg, vg)
                ctx_blocks.append(jnp.concatenate(
                    [ctx_b[0:64], ctx_a, ctx_b[64:128]], axis=0,
                ).astype(jnp.bfloat16))
            ctx_full = jnp.concatenate(ctx_blocks, axis=1)
            return jnp.dot(ctx_full, wo, preferred_element_type=jnp.float32)

        HS = SQ // 2

        def exchange(phase_send, phase_recv, ssems, rsems, partner, b, h):
            return pltpu.make_async_remote_copy(
                src_ref=phase_send.at[b, pl.ds(h * HS, HS)],
                dst_ref=phase_recv.at[b, pl.ds(h * HS, HS)],
                send_sem=ssems.at[b, h],
                recv_sem=rsems.at[b, h],
                device_id=(partner,),
                device_id_type=pl.DeviceIdType.MESH,
            )

        rdma_1 = {}
        rdma_2 = {}
        for b in range(B):
            acc = compute_batch(b)
            out_ref[b] = acc
            send_a[b] = acc.astype(jnp.bfloat16)
            for h in range(2):
                p1 = partner_a if (b + h) % 2 == 0 else partner_b
                rdma_1[b, h] = exchange(send_a, recv_a,
                                        send_sems_a, recv_sems_a, p1, b, h)
                rdma_1[b, h].start()

        for b in range(B):
            for h in range(2):
                rdma_1[b, h].wait()
                hs = slice(h * HS, (h + 1) * HS)
                pair_sum = out_ref[b, hs] + recv_a[b, hs].astype(jnp.float32)
                out_ref[b, hs] = pair_sum
                send_b[b, hs] = pair_sum.astype(jnp.bfloat16)
                p2 = partner_b if (b + h) % 2 == 0 else partner_a
                rdma_2[b, h] = exchange(send_b, recv_b,
                                        send_sems_b, recv_sems_b, p2, b, h)
                rdma_2[b, h].start()

        for b in range(B):
            for h in range(2):
                rdma_2[b, h].wait()
                hs = slice(h * HS, (h + 1) * HS)
                out_ref[b, hs] = out_ref[b, hs] + recv_b[b, hs].astype(jnp.float32)

    comm = pltpu.VMEM((B, SQ, DM), jnp.bfloat16)
    return pl.pallas_call(
        body,
        out_shape=jax.ShapeDtypeStruct((B, SQ, DM), jnp.float32),
        in_specs=[pl.BlockSpec(memory_space=pl.ANY)] * 5,
        out_specs=pl.BlockSpec(memory_space=pltpu.VMEM),
        scratch_shapes=[
            pltpu.VMEM((B, SQ, DM), jnp.float32),
            pltpu.VMEM((DM, HQ_PER * DH), jnp.float32),
            pltpu.VMEM((B, SKV, HQ_PER * DH), jnp.float32),
            pltpu.VMEM((B, SKV, HQ_PER * DH), jnp.float32),
            pltpu.VMEM((HQ_PER * DH, DM), jnp.float32),
            comm, comm, comm, comm,
            pltpu.SemaphoreType.DMA((5,)),
            pltpu.SemaphoreType.DMA((B, 2)),
            pltpu.SemaphoreType.DMA((B, 2)),
            pltpu.SemaphoreType.DMA((B, 2)),
            pltpu.SemaphoreType.DMA((B, 2)),
        ],
        compiler_params=pltpu.CompilerParams(collective_id=0),
    )(x, Wq, k_loc, v_loc, Wo)


# device time: 16323 ns/iter; 1.5664x vs baseline; 1.0074x over previous
import jax
import jax.numpy as jnp
from jax import lax
from jax.experimental import pallas as pl
from jax.experimental.pallas import tpu as pltpu

N_DEV = 4
B = 2
SQ = 256
SKV = 256
HQ_PER = 4
DH = 64
DM = 512


def kernel(x, Wq, K_ext, V_ext, Wo):
    my = lax.axis_index("i")
    k_loc = lax.dynamic_slice_in_dim(
        K_ext, my * HQ_PER, HQ_PER, axis=2
    ).reshape(B, SKV, HQ_PER * DH).astype(jnp.bfloat16)
    v_loc = lax.dynamic_slice_in_dim(
        V_ext, my * HQ_PER, HQ_PER, axis=2
    ).reshape(B, SKV, HQ_PER * DH).astype(jnp.bfloat16)

    def body(x_hbm, wq_hbm, k_hbm, v_hbm, wo_hbm, out_ref,
             xv, wqv, kv, vv, wov,
             send_a, recv_a, send_b, recv_b,
             load_sems, send_sems_a, recv_sems_a, send_sems_b, recv_sems_b):
        my_i = lax.axis_index("i")
        left = (my_i - 1) % N_DEV
        right = (my_i + 1) % N_DEV
        partner_a = my_i ^ 1
        partner_b = 3 - my_i

        loads = [
            pltpu.make_async_copy(x_hbm, xv, load_sems.at[0]),
            pltpu.make_async_copy(wq_hbm, wqv, load_sems.at[1]),
            pltpu.make_async_copy(k_hbm, kv, load_sems.at[2]),
            pltpu.make_async_copy(v_hbm, vv, load_sems.at[3]),
            pltpu.make_async_copy(wo_hbm, wov, load_sems.at[4]),
        ]
        for c in loads:
            c.start()

        barrier = pltpu.get_barrier_semaphore()
        for nbr in (left, right):
            pl.semaphore_signal(
                barrier, inc=1,
                device_id=(nbr,), device_id_type=pl.DeviceIdType.MESH,
            )
        pl.semaphore_wait(barrier, 2)

        for c in loads:
            c.wait()

        wq = (wqv[...] * 0.125).astype(jnp.bfloat16)
        wo = wov[...].astype(jnp.bfloat16)

        def softmax_ctx(q, k, v):
            s = lax.dot_general(
                q, k, (((1,), (1,)), ((), ())),
                preferred_element_type=jnp.float32,
            )
            w = jnp.exp(s)
            r = 1.0 / jnp.sum(w, axis=-1, keepdims=True)
            return jnp.dot((w * r).astype(jnp.bfloat16), v,
                           preferred_element_type=jnp.float32)

        def compute_batch(b):
            xb = xv[b].astype(jnp.bfloat16)
            qf = jnp.dot(xb, wq, preferred_element_type=jnp.float32)
            ctx_blocks = []
            for h in range(HQ_PER):
                qh = qf[:, h * DH:(h + 1) * DH].astype(jnp.bfloat16)
                kh = kv[b][:, h * DH:(h + 1) * DH]
                vh = vv[b][:, h * DH:(h + 1) * DH]
                ctx_a = softmax_ctx(qh[64:192], kh[0:192], vh[0:192])
                qg = jnp.concatenate([qh[0:64], qh[192:256]], axis=0)
                kg = jnp.concatenate([kh[0:64], kh[192:256]], axis=0)
                vg = jnp.concatenate([vh[0:64], vh[192:256]], axis=0)
                ctx_b = softmax_ctx(qg, kg, vg)
                ctx_blocks.append(jnp.concatenate(
                    [ctx_b[0:64], ctx_a, ctx_b[64:128]], axis=0,
                ).astype(jnp.bfloat16))
            ctx_full = jnp.concatenate(ctx_blocks, axis=1)
            return jnp.dot(ctx_full, wo, preferred_element_type=jnp.float32)

        HS = SQ // 2

        def exchange(phase_send, phase_recv, ssems, rsems, partner, b, h):
            return pltpu.make_async_remote_copy(
                src_ref=phase_send.at[b, pl.ds(h * HS, HS)],
                dst_ref=phase_recv.at[b, pl.ds(h * HS, HS)],
                send_sem=ssems.at[b, h],
                recv_sem=rsems.at[b, h],
                device_id=(partner,),
                device_id_type=pl.DeviceIdType.MESH,
            )

        rdma_1 = {}
        rdma_2 = {}
        for b in range(B):
            acc = compute_batch(b)
            out_ref[b] = acc
            send_a[b] = acc.astype(jnp.bfloat16)
            for h in range(2):
                p1 = partner_a if (b + h) % 2 == 0 else partner_b
                rdma_1[b, h] = exchange(send_a, recv_a,
                                        send_sems_a, recv_sems_a, p1, b, h)
                rdma_1[b, h].start()

        for b in range(B):
            for h in range(2):
                rdma_1[b, h].wait()
                hs = slice(h * HS, (h + 1) * HS)
                pair_sum = out_ref[b, hs] + recv_a[b, hs].astype(jnp.float32)
                out_ref[b, hs] = pair_sum
                send_b[b, hs] = pair_sum.astype(jnp.bfloat16)
                p2 = partner_b if (b + h) % 2 == 0 else partner_a
                rdma_2[b, h] = exchange(send_b, recv_b,
                                        send_sems_b, recv_sems_b, p2, b, h)
                rdma_2[b, h].start()

        for b in range(B):
            for h in range(2):
                rdma_2[b, h].wait()
                hs = slice(h * HS, (h + 1) * HS)
                out_ref[b, hs] = out_ref[b, hs] + recv_b[b, hs].astype(jnp.float32)

    comm = pltpu.VMEM((B, SQ, DM), jnp.bfloat16)
    return pl.pallas_call(
        body,
        out_shape=jax.ShapeDtypeStruct((B, SQ, DM), jnp.float32),
        in_specs=[pl.BlockSpec(memory_space=pl.ANY)] * 5,
        out_specs=pl.BlockSpec(memory_space=pltpu.VMEM),
        scratch_shapes=[
            pltpu.VMEM((B, SQ, DM), jnp.float32),
            pltpu.VMEM((DM, HQ_PER * DH), jnp.float32),
            pltpu.VMEM((B, SKV, HQ_PER * DH), jnp.bfloat16),
            pltpu.VMEM((B, SKV, HQ_PER * DH), jnp.bfloat16),
            pltpu.VMEM((HQ_PER * DH, DM), jnp.float32),
            comm, comm, comm, comm,
            pltpu.SemaphoreType.DMA((5,)),
            pltpu.SemaphoreType.DMA((B, 2)),
            pltpu.SemaphoreType.DMA((B, 2)),
            pltpu.SemaphoreType.DMA((B, 2)),
            pltpu.SemaphoreType.DMA((B, 2)),
        ],
        compiler_params=pltpu.CompilerParams(collective_id=0),
    )(x, Wq, k_loc, v_loc, Wo)


# device time: 15923 ns/iter; 1.6057x vs baseline; 1.0251x over previous
import jax
import jax.numpy as jnp
from jax import lax
from jax.experimental import pallas as pl
from jax.experimental.pallas import tpu as pltpu

N_DEV = 4
B = 2
SQ = 256
SKV = 256
HQ_PER = 4
DH = 64
DM = 512


def kernel(x, Wq, K_ext, V_ext, Wo):
    my = lax.axis_index("i")
    k_loc = lax.dynamic_slice_in_dim(
        K_ext, my * HQ_PER, HQ_PER, axis=2
    ).reshape(B, SKV, HQ_PER * DH).astype(jnp.bfloat16)
    v_loc = lax.dynamic_slice_in_dim(
        V_ext, my * HQ_PER, HQ_PER, axis=2
    ).reshape(B, SKV, HQ_PER * DH).astype(jnp.bfloat16)

    def body(x_hbm, wq_hbm, k_hbm, v_hbm, wo_hbm, out_ref,
             xv, wqv, kv, vv, wov, accv,
             send_a, recv_a, send_b, recv_b,
             load_sems, store_sem,
             send_sems_a, recv_sems_a, send_sems_b, recv_sems_b):
        my_i = lax.axis_index("i")
        left = (my_i - 1) % N_DEV
        right = (my_i + 1) % N_DEV
        partner_a = my_i ^ 1
        partner_b = 3 - my_i

        loads = [
            pltpu.make_async_copy(x_hbm, xv, load_sems.at[0]),
            pltpu.make_async_copy(wq_hbm, wqv, load_sems.at[1]),
            pltpu.make_async_copy(k_hbm, kv, load_sems.at[2]),
            pltpu.make_async_copy(v_hbm, vv, load_sems.at[3]),
            pltpu.make_async_copy(wo_hbm, wov, load_sems.at[4]),
        ]
        for c in loads:
            c.start()

        barrier = pltpu.get_barrier_semaphore()
        for nbr in (left, right):
            pl.semaphore_signal(
                barrier, inc=1,
                device_id=(nbr,), device_id_type=pl.DeviceIdType.MESH,
            )
        pl.semaphore_wait(barrier, 2)

        for c in loads:
            c.wait()

        wq = (wqv[...] * 0.125).astype(jnp.bfloat16)
        wo = wov[...].astype(jnp.bfloat16)

        def softmax_ctx(q, k, v):
            s = lax.dot_general(
                q, k, (((1,), (1,)), ((), ())),
                preferred_element_type=jnp.float32,
            )
            w = jnp.exp(s)
            r = 1.0 / jnp.sum(w, axis=-1, keepdims=True)
            return jnp.dot((w * r).astype(jnp.bfloat16), v,
                           preferred_element_type=jnp.float32)

        def compute_batch(b):
            xb = xv[b].astype(jnp.bfloat16)
            qf = jnp.dot(xb, wq, preferred_element_type=jnp.float32)
            ctx_blocks = []
            for h in range(HQ_PER):
                qh = qf[:, h * DH:(h + 1) * DH].astype(jnp.bfloat16)
                kh = kv[b][:, h * DH:(h + 1) * DH]
                vh = vv[b][:, h * DH:(h + 1) * DH]
                ctx_a = softmax_ctx(qh[64:192], kh[0:192], vh[0:192])
                qg = jnp.concatenate([qh[0:64], qh[192:256]], axis=0)
                kg = jnp.concatenate([kh[0:64], kh[192:256]], axis=0)
                vg = jnp.concatenate([vh[0:64], vh[192:256]], axis=0)
                ctx_b = softmax_ctx(qg, kg, vg)
                ctx_blocks.append(jnp.concatenate(
                    [ctx_b[0:64], ctx_a, ctx_b[64:128]], axis=0,
                ).astype(jnp.bfloat16))
            ctx_full = jnp.concatenate(ctx_blocks, axis=1)
            return jnp.dot(ctx_full, wo, preferred_element_type=jnp.float32)

        HS = SQ // 2

        def exchange(phase_send, phase_recv, ssems, rsems, partner, b, h):
            return pltpu.make_async_remote_copy(
                src_ref=phase_send.at[b, pl.ds(h * HS, HS)],
                dst_ref=phase_recv.at[b, pl.ds(h * HS, HS)],
                send_sem=ssems.at[b, h],
                recv_sem=rsems.at[b, h],
                device_id=(partner,),
                device_id_type=pl.DeviceIdType.MESH,
            )

        rdma_1 = {}
        rdma_2 = {}
        for b in range(B):
            acc = compute_batch(b)
            accv[b] = acc
            send_a[b] = acc.astype(jnp.bfloat16)
            for h in range(2):
                p1 = partner_a if (b + h) % 2 == 0 else partner_b
                rdma_1[b, h] = exchange(send_a, recv_a,
                                        send_sems_a, recv_sems_a, p1, b, h)
                rdma_1[b, h].start()

        for b in range(B):
            for h in range(2):
                rdma_1[b, h].wait()
                hs = slice(h * HS, (h + 1) * HS)
                pair_sum = accv[b, hs] + recv_a[b, hs].astype(jnp.float32)
                accv[b, hs] = pair_sum
                send_b[b, hs] = pair_sum.astype(jnp.bfloat16)
                p2 = partner_b if (b + h) % 2 == 0 else partner_a
                rdma_2[b, h] = exchange(send_b, recv_b,
                                        send_sems_b, recv_sems_b, p2, b, h)
                rdma_2[b, h].start()

        for b in range(B):
            for h in range(2):
                rdma_2[b, h].wait()
                hs = slice(h * HS, (h + 1) * HS)
                accv[b, hs] = accv[b, hs] + recv_b[b, hs].astype(jnp.float32)

        store = pltpu.make_async_copy(accv, out_ref, store_sem)
        store.start()
        store.wait()

    comm = pltpu.VMEM((B, SQ, DM), jnp.bfloat16)
    return pl.pallas_call(
        body,
        out_shape=jax.ShapeDtypeStruct((B, SQ, DM), jnp.float32),
        in_specs=[pl.BlockSpec(memory_space=pl.ANY)] * 5,
        out_specs=pl.BlockSpec(memory_space=pl.ANY),
        scratch_shapes=[
            pltpu.VMEM((B, SQ, DM), jnp.float32),
            pltpu.VMEM((DM, HQ_PER * DH), jnp.float32),
            pltpu.VMEM((B, SKV, HQ_PER * DH), jnp.bfloat16),
            pltpu.VMEM((B, SKV, HQ_PER * DH), jnp.bfloat16),
            pltpu.VMEM((HQ_PER * DH, DM), jnp.float32),
            pltpu.VMEM((B, SQ, DM), jnp.float32),
            comm, comm, comm, comm,
            pltpu.SemaphoreType.DMA((5,)),
            pltpu.SemaphoreType.DMA,
            pltpu.SemaphoreType.DMA((B, 2)),
            pltpu.SemaphoreType.DMA((B, 2)),
            pltpu.SemaphoreType.DMA((B, 2)),
            pltpu.SemaphoreType.DMA((B, 2)),
        ],
        compiler_params=pltpu.CompilerParams(collective_id=0),
    )(x, Wq, k_loc, v_loc, Wo)
